# Initial kernel scaffold; baseline (speedup 1.0000x reference)
import jax
import jax.numpy as jnp
from jax.experimental import pallas as pl


def kernel(x, adj_indices, adj_values, weight, bias):
    raise NotImplementedError("write your pallas kernel here")


# speedup vs baseline: 3.6910x; 3.6910x over previous
import functools

import jax
import jax.numpy as jnp
from jax import lax
from jax.experimental import pallas as pl
from jax.experimental.pallas import tpu as pltpu
from jax.experimental.pallas import tpu_sc as plsc

NC = 2
NS = 16
LANES = 16
B = 128
F = 128


def _tc_linear(x, weight, bias):
    n, f_in = x.shape
    f_out = weight.shape[1]
    blk = 1000

    def mm_kernel(x_ref, w_ref, b_ref, o_ref):
        o_ref[...] = jnp.dot(
            x_ref[...], w_ref[...],
            preferred_element_type=jnp.float32,
            precision=lax.Precision.HIGHEST,
        ) + b_ref[...]

    return pl.pallas_call(
        mm_kernel,
        grid=(n // blk,),
        in_specs=[
            pl.BlockSpec((blk, f_in), lambda i: (i, 0)),
            pl.BlockSpec((f_in, f_out), lambda i: (0, 0)),
            pl.BlockSpec((1, f_out), lambda i: (0, 0)),
        ],
        out_specs=pl.BlockSpec((blk, f_out), lambda i: (i, 0)),
        out_shape=jax.ShapeDtypeStruct((n, f_out), jnp.float32),
    )(x, weight, bias.reshape(1, f_out))


def _tc_add(partials):
    _, n, f = partials.shape
    blk = 1000

    def add_kernel(p_ref, o_ref):
        o_ref[...] = p_ref[0] + p_ref[1]

    return pl.pallas_call(
        add_kernel,
        grid=(n // blk,),
        in_specs=[pl.BlockSpec((2, blk, f), lambda i: (0, i, 0))],
        out_specs=pl.BlockSpec((blk, f), lambda i: (i, 0)),
        out_shape=jax.ShapeDtypeStruct((n, f), jnp.float32),
    )(partials)


def _sc_aggregate(h, row, col, val, n_nodes):
    e_pad = row.shape[0]
    edges_per_tile = e_pad // (NC * NS)
    n_batches = edges_per_tile // B
    rows_per_sub = n_nodes // NS
    full = rows_per_sub // B
    rem = rows_per_sub - full * B
    mesh = plsc.VectorSubcoreMesh(core_axis_name="c", subcore_axis_name="s")

    @functools.partial(
        pl.kernel,
        out_type=jax.ShapeDtypeStruct((NC, NS, rows_per_sub, F), jnp.float32),
        mesh=mesh,
        scratch_types=[
            pltpu.VMEM((B,), jnp.int32),
            pltpu.VMEM((B,), jnp.int32),
            pltpu.VMEM((B,), jnp.float32),
            pltpu.VMEM((B, F), jnp.float32),
            pltpu.VMEM_SHARED((n_nodes, F), jnp.float32),
        ],
    )
    def sc_kernel(h_hbm, row_hbm, col_hbm, val_hbm, out_hbm,
                  col_v, row_v, val_v, gbuf, acc):
        cid = lax.axis_index("c")
        sid = lax.axis_index("s")
        wid = sid * NC + cid
        base = wid * edges_per_tile
        rbase = sid * rows_per_sub

        zeros16 = jnp.zeros((LANES,), jnp.float32)

        @pl.loop(0, B)
        def _(i):
            @pl.loop(0, F, step=LANES)
            def _(c):
                gbuf[i, pl.ds(c, LANES)] = zeros16

        for k in range(full):
            pltpu.sync_copy(gbuf, acc.at[pl.ds(rbase + k * B, B)])
        if rem:
            pltpu.sync_copy(gbuf.at[pl.ds(0, rem)],
                            acc.at[pl.ds(rbase + full * B, rem)])
        plsc.subcore_barrier()

        @pl.loop(0, n_batches)
        def _(j):
            off = base + j * B
            pltpu.sync_copy(col_hbm.at[pl.ds(off, B)], col_v)
            pltpu.sync_copy(row_hbm.at[pl.ds(off, B)], row_v)
            pltpu.sync_copy(val_hbm.at[pl.ds(off, B)], val_v)
            pltpu.sync_copy(h_hbm.at[col_v], gbuf)

            @pl.loop(0, B, step=LANES)
            def _(i0):
                v16 = val_v[pl.ds(i0, LANES)]
                for r in range(LANES):
                    bc = jnp.full((LANES,), v16[r], jnp.float32)
                    for c in range(0, F, LANES):
                        gbuf[i0 + r, pl.ds(c, LANES)] = (
                            gbuf[i0 + r, pl.ds(c, LANES)] * bc)

            pltpu.sync_copy(gbuf, acc.at[row_v], add=True)

        plsc.subcore_barrier()
        pltpu.sync_copy(acc.at[pl.ds(rbase, rows_per_sub)],
                        out_hbm.at[cid, sid])

    return sc_kernel(h, row, col, val).reshape(NC, n_nodes, F)


def kernel(x, adj_indices, adj_values, weight, bias):
    n_nodes = x.shape[0]
    row = adj_indices[0].astype(jnp.int32)
    col = adj_indices[1].astype(jnp.int32)
    val = adj_values.astype(jnp.float32)
    e = row.shape[0]
    tile_e = NC * NS * B
    e_pad = ((e + tile_e - 1) // tile_e) * tile_e
    if e_pad != e:
        pad = e_pad - e
        row = jnp.concatenate([row, jnp.zeros((pad,), jnp.int32)])
        col = jnp.concatenate([col, jnp.zeros((pad,), jnp.int32)])
        val = jnp.concatenate([val, jnp.zeros((pad,), jnp.float32)])

    h = _tc_linear(x, weight, bias)
    partials = _sc_aggregate(h, row, col, val, n_nodes)
    return _tc_add(partials)


# speedup vs baseline: 4.0179x; 1.0886x over previous
import dataclasses
import functools

import jax
import jax.numpy as jnp
from jax import lax
from jax.experimental import pallas as pl
from jax.experimental.pallas import tpu as pltpu
from jax.experimental.pallas import tpu_sc as plsc

NC = 2
NS = 16
LANES = 16
B = 128
F = 128


def _tc_linear(x, weight, bias):
    n, f_in = x.shape
    f_out = weight.shape[1]
    blk = 1000

    def mm_kernel(x_ref, w_ref, b_ref, o_ref):
        o_ref[...] = jnp.dot(
            x_ref[...], w_ref[...],
            preferred_element_type=jnp.float32,
            precision=lax.Precision.HIGHEST,
        ) + b_ref[...]

    return pl.pallas_call(
        mm_kernel,
        grid=(n // blk,),
        in_specs=[
            pl.BlockSpec((blk, f_in), lambda i: (i, 0)),
            pl.BlockSpec((f_in, f_out), lambda i: (0, 0)),
            pl.BlockSpec((1, f_out), lambda i: (0, 0)),
        ],
        out_specs=pl.BlockSpec((blk, f_out), lambda i: (i, 0)),
        out_shape=jax.ShapeDtypeStruct((n, f_out), jnp.float32),
    )(x, weight, bias.reshape(1, f_out))


def _tc_add(partials):
    _, n, f = partials.shape
    blk = 1000

    def add_kernel(p_ref, o_ref):
        o_ref[...] = p_ref[0] + p_ref[1]

    return pl.pallas_call(
        add_kernel,
        grid=(n // blk,),
        in_specs=[pl.BlockSpec((2, blk, f), lambda i: (0, i, 0))],
        out_specs=pl.BlockSpec((blk, f), lambda i: (i, 0)),
        out_shape=jax.ShapeDtypeStruct((n, f), jnp.float32),
    )(partials)


def _sc_aggregate(h, edges, n_nodes):
    e_pad = edges.shape[1]
    edges_per_tile = e_pad // (NC * NS)
    n_batches = edges_per_tile // B
    rows_per_sub = n_nodes // NS
    full = rows_per_sub // B
    rem = rows_per_sub - full * B
    mesh = plsc.VectorSubcoreMesh(core_axis_name="c", subcore_axis_name="s")
    cp = pltpu.CompilerParams()
    if "needs_layout_passes" in pltpu.CompilerParams.__dataclass_fields__:
        cp = dataclasses.replace(cp, needs_layout_passes=False)

    @functools.partial(
        pl.kernel,
        out_type=jax.ShapeDtypeStruct((NC, NS, rows_per_sub, F), jnp.float32),
        mesh=mesh,
        compiler_params=cp,
        scratch_types=[
            pltpu.VMEM((3, B), jnp.int32),
            pltpu.VMEM((3, B), jnp.int32),
            pltpu.VMEM((B, F), jnp.float32),
            pltpu.VMEM((B, F), jnp.float32),
            pltpu.VMEM_SHARED((n_nodes, F), jnp.float32),
            pltpu.SemaphoreType.DMA,
            pltpu.SemaphoreType.DMA,
        ],
    )
    def sc_kernel(h_hbm, e_hbm, out_hbm, eb0, eb1, gb0, gb1, acc, gs0, gs1):
        cid = lax.axis_index("c")
        sid = lax.axis_index("s")
        wid = sid * NC + cid
        base = wid * edges_per_tile
        rbase = sid * rows_per_sub
        ebufs = (eb0, eb1)
        gbufs = (gb0, gb1)
        gsems = (gs0, gs1)

        zeros16 = jnp.zeros((LANES,), jnp.float32)

        @pl.loop(0, B)
        def _(i):
            @pl.loop(0, F, step=LANES)
            def _(c):
                gb0[i, pl.ds(c, LANES)] = zeros16

        for k in range(full):
            pltpu.sync_copy(gb0, acc.at[pl.ds(rbase + k * B, B)])
        if rem:
            pltpu.sync_copy(gb0.at[pl.ds(0, rem)],
                            acc.at[pl.ds(rbase + full * B, rem)])
        plsc.subcore_barrier()

        def fetch(b, p):
            pltpu.sync_copy(e_hbm.at[:, pl.ds(base + b * B, B)], ebufs[p])
            pltpu.async_copy(h_hbm.at[ebufs[p].at[1]], gbufs[p], gsems[p])

        fetch(0, 0)

        def body(b, p):
            nb = b + 1

            @pl.when(nb < n_batches)
            def _():
                fetch(nb, 1 - p)

            pltpu.make_async_copy(h_hbm.at[ebufs[p].at[1]], gbufs[p],
                                  gsems[p]).wait()
            gbuf = gbufs[p]
            ebuf = ebufs[p]

            @pl.loop(0, B, step=LANES)
            def _(i0):
                v16 = plsc.bitcast(ebuf[2, pl.ds(i0, LANES)], jnp.float32)
                for r in range(LANES):
                    bc = jnp.full((LANES,), v16[r], jnp.float32)
                    for c in range(0, F, LANES):
                        gbuf[i0 + r, pl.ds(c, LANES)] = (
                            gbuf[i0 + r, pl.ds(c, LANES)] * bc)

            pltpu.sync_copy(gbuf, acc.at[ebuf.at[0]], add=True)

        @pl.loop(0, n_batches, step=2)
        def _(j):
            body(j, 0)
            body(j + 1, 1)

        plsc.subcore_barrier()
        pltpu.sync_copy(acc.at[pl.ds(rbase, rows_per_sub)],
                        out_hbm.at[cid, sid])

    return sc_kernel(h, edges).reshape(NC, n_nodes, F)


def kernel(x, adj_indices, adj_values, weight, bias):
    n_nodes = x.shape[0]
    row = adj_indices[0].astype(jnp.int32)
    col = adj_indices[1].astype(jnp.int32)
    val = adj_values.astype(jnp.float32)
    e = row.shape[0]
    tile_e = NC * NS * B * 2
    e_pad = ((e + tile_e - 1) // tile_e) * tile_e
    if e_pad != e:
        pad = e_pad - e
        row = jnp.concatenate([row, jnp.zeros((pad,), jnp.int32)])
        col = jnp.concatenate([col, jnp.zeros((pad,), jnp.int32)])
        val = jnp.concatenate([val, jnp.zeros((pad,), jnp.float32)])
    edges = jnp.stack(
        [row, col, lax.bitcast_convert_type(val, jnp.int32)])

    h = _tc_linear(x, weight, bias)
    partials = _sc_aggregate(h, edges, n_nodes)
    return _tc_add(partials)


# speedup vs baseline: 4.0238x; 1.0015x over previous
import dataclasses
import functools

import jax
import jax.numpy as jnp
from jax import lax
from jax.experimental import pallas as pl
from jax.experimental.pallas import tpu as pltpu
from jax.experimental.pallas import tpu_sc as plsc

NC = 2
NS = 16
LANES = 16
B = 128
F = 128


def _tc_linear(x, weight, bias):
    n, f_in = x.shape
    f_out = weight.shape[1]
    blk = 1000

    def mm_kernel(x_ref, w_ref, b_ref, o_ref):
        o_ref[...] = jnp.dot(
            x_ref[...], w_ref[...],
            preferred_element_type=jnp.float32,
            precision=lax.Precision.HIGHEST,
        ) + b_ref[...]

    return pl.pallas_call(
        mm_kernel,
        grid=(n // blk,),
        in_specs=[
            pl.BlockSpec((blk, f_in), lambda i: (i, 0)),
            pl.BlockSpec((f_in, f_out), lambda i: (0, 0)),
            pl.BlockSpec((1, f_out), lambda i: (0, 0)),
        ],
        out_specs=pl.BlockSpec((blk, f_out), lambda i: (i, 0)),
        out_shape=jax.ShapeDtypeStruct((n, f_out), jnp.float32),
    )(x, weight, bias.reshape(1, f_out))


def _tc_add(partials):
    _, n, f = partials.shape
    blk = 1000

    def add_kernel(p_ref, o_ref):
        o_ref[...] = p_ref[0] + p_ref[1]

    return pl.pallas_call(
        add_kernel,
        grid=(n // blk,),
        in_specs=[pl.BlockSpec((2, blk, f), lambda i: (0, i, 0))],
        out_specs=pl.BlockSpec((blk, f), lambda i: (i, 0)),
        out_shape=jax.ShapeDtypeStruct((n, f), jnp.float32),
    )(partials)


def _sc_aggregate(h, edges, n_nodes):
    e_pad = edges.shape[1]
    edges_per_tile = e_pad // (NC * NS)
    n_batches = edges_per_tile // B
    rows_per_sub = n_nodes // NS
    full = rows_per_sub // B
    rem = rows_per_sub - full * B
    mesh = plsc.VectorSubcoreMesh(core_axis_name="c", subcore_axis_name="s")
    cp = pltpu.CompilerParams()
    if "needs_layout_passes" in pltpu.CompilerParams.__dataclass_fields__:
        cp = dataclasses.replace(cp, needs_layout_passes=False)

    @functools.partial(
        pl.kernel,
        out_type=jax.ShapeDtypeStruct((NC, NS, rows_per_sub, F), jnp.float32),
        mesh=mesh,
        compiler_params=cp,
        scratch_types=[
            pltpu.VMEM((3, B), jnp.int32),
            pltpu.VMEM((3, B), jnp.int32),
            pltpu.VMEM((B, F), jnp.float32),
            pltpu.VMEM((B, F), jnp.float32),
            pltpu.VMEM_SHARED((n_nodes, F), jnp.float32),
            pltpu.SemaphoreType.DMA,
            pltpu.SemaphoreType.DMA,
            pltpu.SemaphoreType.DMA,
            pltpu.SemaphoreType.DMA,
        ],
    )
    def sc_kernel(h_hbm, e_hbm, out_hbm, eb0, eb1, gb0, gb1, acc,
                  gs0, gs1, ss0, ss1):
        cid = lax.axis_index("c")
        sid = lax.axis_index("s")
        wid = sid * NC + cid
        base = wid * edges_per_tile
        rbase = sid * rows_per_sub
        ebufs = (eb0, eb1)
        gbufs = (gb0, gb1)
        gsems = (gs0, gs1)
        ssems = (ss0, ss1)

        def fetch(b, p):
            pltpu.sync_copy(e_hbm.at[:, pl.ds(base + b * B, B)], ebufs[p])
            pltpu.async_copy(h_hbm.at[ebufs[p].at[1]], gbufs[p], gsems[p])

        fetch(0, 0)

        zeros16 = jnp.zeros((LANES,), jnp.float32)

        @pl.loop(0, B)
        def _(i):
            @pl.loop(0, F, step=LANES)
            def _(c):
                gb1[i, pl.ds(c, LANES)] = zeros16

        for k in range(full):
            pltpu.sync_copy(gb1, acc.at[pl.ds(rbase + k * B, B)])
        if rem:
            pltpu.sync_copy(gb1.at[pl.ds(0, rem)],
                            acc.at[pl.ds(rbase + full * B, rem)])
        plsc.subcore_barrier()

        def body(b, p):
            @pl.when(b >= 1)
            def _():
                pltpu.make_async_copy(gbufs[1 - p],
                                      acc.at[ebufs[1 - p].at[0]],
                                      ssems[1 - p]).wait()

            @pl.when(b + 1 < n_batches)
            def _():
                fetch(b + 1, 1 - p)

            pltpu.make_async_copy(h_hbm.at[ebufs[p].at[1]], gbufs[p],
                                  gsems[p]).wait()
            gbuf = gbufs[p]
            ebuf = ebufs[p]

            @pl.loop(0, B, step=LANES)
            def _(i0):
                v16 = plsc.bitcast(ebuf[2, pl.ds(i0, LANES)], jnp.float32)
                for r in range(LANES):
                    bc = jnp.full((LANES,), v16[r], jnp.float32)
                    for c in range(0, F, LANES):
                        gbuf[i0 + r, pl.ds(c, LANES)] = (
                            gbuf[i0 + r, pl.ds(c, LANES)] * bc)

            pltpu.async_copy(gbuf, acc.at[ebuf.at[0]], ssems[p], add=True)

        @pl.loop(0, n_batches, step=2)
        def _(j):
            body(j, 0)
            body(j + 1, 1)

        pltpu.make_async_copy(gb1, acc.at[eb1.at[0]], ss1).wait()
        plsc.subcore_barrier()
        pltpu.sync_copy(acc.at[pl.ds(rbase, rows_per_sub)],
                        out_hbm.at[cid, sid])

    return sc_kernel(h, edges).reshape(NC, n_nodes, F)


def kernel(x, adj_indices, adj_values, weight, bias):
    n_nodes = x.shape[0]
    row = adj_indices[0].astype(jnp.int32)
    col = adj_indices[1].astype(jnp.int32)
    val = adj_values.astype(jnp.float32)
    e = row.shape[0]
    tile_e = NC * NS * B * 2
    e_pad = ((e + tile_e - 1) // tile_e) * tile_e
    if e_pad != e:
        pad = e_pad - e
        row = jnp.concatenate([row, jnp.zeros((pad,), jnp.int32)])
        col = jnp.concatenate([col, jnp.zeros((pad,), jnp.int32)])
        val = jnp.concatenate([val, jnp.zeros((pad,), jnp.float32)])
    edges = jnp.stack(
        [row, col, lax.bitcast_convert_type(val, jnp.int32)])

    h = _tc_linear(x, weight, bias)
    partials = _sc_aggregate(h, edges, n_nodes)
    return _tc_add(partials)
